# baseline (speedup 1.0000x reference)
import functools

import jax
import jax.numpy as jnp
from jax import lax
from jax.experimental import pallas as pl
from jax.experimental.pallas import tpu as pltpu
from jax.experimental.pallas import tpu_sc as plsc

NC = 2
NS = 16
L = 16
NW = NC * NS

CHUNK = 2048
NB = 3


@functools.cache
def _sc_edge_kernel(n_nodes: int, n_acc: int, e_pad: int, chunks_per_tile: int):
    edges_per_tile = e_pad // NW
    zslice = n_acc // NS

    mesh = plsc.VectorSubcoreMesh(core_axis_name="c", subcore_axis_name="s")

    idx_buf = pltpu.VMEM((CHUNK,), jnp.int32)
    val_buf = pltpu.VMEM((CHUNK,), jnp.float32)

    @functools.partial(
        pl.kernel,
        mesh=mesh,
        compiler_params=pltpu.CompilerParams(needs_layout_passes=False),
        out_type=jax.ShapeDtypeStruct((NC, n_acc), jnp.float32),
        scratch_types=[
            pltpu.VMEM((n_nodes,), jnp.float32),
            [idx_buf] * NB,
            [idx_buf] * NB,
            [val_buf] * NB,
            [val_buf] * NB,
            pltpu.VMEM_SHARED((n_acc,), jnp.float32),
            [pltpu.SemaphoreType.DMA] * NB,
            [pltpu.SemaphoreType.DMA] * NB,
        ],
    )
    def sc_kernel(nodes_hbm, row_hbm, col_hbm, ev_hbm, zeros_hbm, out_hbm,
                  nodes_v, rbufs, cbufs, ebufs, lbufs, acc_sh,
                  in_sems, sc_sems):
        c = lax.axis_index("c")
        s = lax.axis_index("s")
        wid = s * NC + c
        base_edge = wid * edges_per_tile

        def in_descs(ci, slot):
            sl = pl.ds(base_edge + ci * CHUNK, CHUNK)
            sem = in_sems[slot]
            return (
                pltpu.make_async_copy(row_hbm.at[sl], rbufs[slot], sem),
                pltpu.make_async_copy(col_hbm.at[sl], cbufs[slot], sem),
                pltpu.make_async_copy(ev_hbm.at[sl], ebufs[slot], sem),
            )

        def sc_desc(slot):
            return pltpu.make_async_copy(
                lbufs[slot], acc_sh.at[cbufs[slot]], sc_sems[slot])

        pltpu.sync_copy(nodes_hbm, nodes_v)
        pltpu.sync_copy(zeros_hbm.at[pl.ds(s * zslice, zslice)],
                        acc_sh.at[pl.ds(s * zslice, zslice)])
        plsc.subcore_barrier()

        for d in in_descs(0, 0):
            d.start()

        @pl.loop(0, chunks_per_tile, step=NB)
        def _group(bi):
            for p in range(NB):
                ci = bi + p
                dslot = (p + 1) % NB
                if p == NB - 1:
                    sc_desc(dslot).wait()
                else:

                    @pl.when(bi >= NB)
                    def _():
                        sc_desc(dslot).wait()

                if p == NB - 1:

                    @pl.when(bi + NB < chunks_per_tile)
                    def _():
                        for d in in_descs(ci + 1, dslot):
                            d.start()
                else:
                    for d in in_descs(ci + 1, dslot):
                        d.start()

                for d in in_descs(ci, p):
                    d.wait()
                for k in range(CHUNK // L):
                    sl = pl.ds(k * L, L)
                    ir = rbufs[p][sl]
                    ic = cbufs[p][sl]
                    e = ebufs[p][sl]
                    src = plsc.load_gather(nodes_v, [ir])
                    dst = plsc.load_gather(nodes_v, [ic])
                    m = (src * e) > 0
                    safe = jnp.where(m, e, jnp.float32(1.0))
                    lbufs[p][sl] = jnp.where(m, (dst - src) / safe,
                                             jnp.float32(0.0))
                sc_desc(p).start(add=True)

        sc_desc(1).wait()
        sc_desc(2).wait()

        plsc.subcore_barrier()

        @pl.when(s == 0)
        def _():
            pltpu.sync_copy(acc_sh, out_hbm.at[c])

    return sc_kernel


@functools.cache
def _tc_sum_kernel(n_acc: int):
    def body(p_ref, o_ref):
        o_ref[...] = p_ref[0] + p_ref[1]

    return pl.pallas_call(
        body,
        out_shape=jax.ShapeDtypeStruct((n_acc // 128, 128), jnp.float32),
    )


def kernel(x, edge_index, edge_attr):
    n = x.shape[0]
    e_cnt = edge_index.shape[1]

    nodes = x[:, 0]
    row = edge_index[0].astype(jnp.int32)
    col = edge_index[1].astype(jnp.int32)
    ev = edge_attr[:, 0]

    grain = NW * CHUNK * NB
    e_pad = -(-e_cnt // grain) * grain
    pad = e_pad - e_cnt
    if pad:
        row = jnp.concatenate([row, jnp.zeros((pad,), jnp.int32)])
        col = jnp.concatenate([col, jnp.zeros((pad,), jnp.int32)])
        ev = jnp.concatenate([ev, jnp.zeros((pad,), jnp.float32)])

    n_acc = -(-n // 2048) * 2048
    sc = _sc_edge_kernel(n, n_acc, e_pad, e_pad // (NW * CHUNK))
    partial = sc(nodes, row, col, ev, jnp.zeros((n_acc,), jnp.float32))

    summed = _tc_sum_kernel(n_acc)(partial.reshape(NC, n_acc // 128, 128))
    return summed.reshape(-1)[:n]


# speedup vs baseline: 1.1434x; 1.1434x over previous
import functools

import jax
import jax.numpy as jnp
from jax import lax
from jax.experimental import pallas as pl
from jax.experimental.pallas import tpu as pltpu
from jax.experimental.pallas import tpu_sc as plsc

NC = 2
NS = 16
L = 16
NW = NC * NS

CHUNK = 1024
NB = 6
DEPTH = 3


@functools.cache
def _sc_edge_kernel(n_nodes: int, n_acc: int, e_pad: int, chunks_per_tile: int):
    edges_per_tile = e_pad // NW
    zslice = n_acc // NS

    mesh = plsc.VectorSubcoreMesh(core_axis_name="c", subcore_axis_name="s")

    idx_buf = pltpu.VMEM((CHUNK,), jnp.int32)
    val_buf = pltpu.VMEM((CHUNK,), jnp.float32)

    @functools.partial(
        pl.kernel,
        mesh=mesh,
        compiler_params=pltpu.CompilerParams(needs_layout_passes=False),
        out_type=jax.ShapeDtypeStruct((NC, n_acc), jnp.float32),
        scratch_types=[
            pltpu.VMEM((n_nodes,), jnp.float32),
            [idx_buf] * NB,
            [idx_buf] * NB,
            [val_buf] * NB,
            [val_buf] * NB,
            pltpu.VMEM_SHARED((n_acc,), jnp.float32),
            [pltpu.SemaphoreType.DMA] * NB,
            [pltpu.SemaphoreType.DMA] * NB,
        ],
    )
    def sc_kernel(nodes_hbm, row_hbm, col_hbm, ev_hbm, zeros_hbm, out_hbm,
                  nodes_v, rbufs, cbufs, ebufs, lbufs, acc_sh,
                  in_sems, sc_sems):
        c = lax.axis_index("c")
        s = lax.axis_index("s")
        wid = s * NC + c
        base_edge = wid * edges_per_tile

        def in_descs(ci, slot):
            sl = pl.ds(base_edge + ci * CHUNK, CHUNK)
            sem = in_sems[slot]
            return (
                pltpu.make_async_copy(row_hbm.at[sl], rbufs[slot], sem),
                pltpu.make_async_copy(col_hbm.at[sl], cbufs[slot], sem),
                pltpu.make_async_copy(ev_hbm.at[sl], ebufs[slot], sem),
            )

        def sc_desc(slot):
            return pltpu.make_async_copy(
                lbufs[slot], acc_sh.at[cbufs[slot]], sc_sems[slot])

        pltpu.sync_copy(nodes_hbm, nodes_v)
        pltpu.sync_copy(zeros_hbm.at[pl.ds(s * zslice, zslice)],
                        acc_sh.at[pl.ds(s * zslice, zslice)])
        plsc.subcore_barrier()

        for ci in range(DEPTH):
            for d in in_descs(ci, ci % NB):
                d.start()

        @pl.loop(0, chunks_per_tile, step=NB)
        def _group(bi):
            for p in range(NB):
                ci = bi + p
                fslot = (p + DEPTH) % NB
                if p >= NB - DEPTH:
                    sc_desc(fslot).wait()

                    @pl.when(ci + DEPTH < chunks_per_tile)
                    def _():
                        for d in in_descs(ci + DEPTH, fslot):
                            d.start()
                else:
                    @pl.when(bi > 0)
                    def _():
                        sc_desc(fslot).wait()

                    for d in in_descs(ci + DEPTH, fslot):
                        d.start()

                for d in in_descs(ci, p):
                    d.wait()
                for k in range(CHUNK // L):
                    sl = pl.ds(k * L, L)
                    ir = rbufs[p][sl]
                    ic = cbufs[p][sl]
                    e = ebufs[p][sl]
                    src = plsc.load_gather(nodes_v, [ir])
                    dst = plsc.load_gather(nodes_v, [ic])
                    m = (src * e) > 0
                    lbufs[p][sl] = jnp.where(m, (dst - src) / e,
                                             jnp.float32(0.0))
                sc_desc(p).start(add=True)

        for q in range(NB - DEPTH, NB):
            sc_desc(q % NB).wait()

        plsc.subcore_barrier()

        @pl.when(s == 0)
        def _():
            pltpu.sync_copy(acc_sh, out_hbm.at[c])

    return sc_kernel


@functools.cache
def _tc_sum_kernel(n_acc: int):
    def body(p_ref, o_ref):
        o_ref[...] = p_ref[0] + p_ref[1]

    return pl.pallas_call(
        body,
        out_shape=jax.ShapeDtypeStruct((n_acc // 128, 128), jnp.float32),
    )


def kernel(x, edge_index, edge_attr):
    n = x.shape[0]
    e_cnt = edge_index.shape[1]

    nodes = x[:, 0]
    row = edge_index[0].astype(jnp.int32)
    col = edge_index[1].astype(jnp.int32)
    ev = edge_attr[:, 0]

    grain = NW * CHUNK * NB
    e_pad = -(-e_cnt // grain) * grain
    pad = e_pad - e_cnt
    if pad:
        row = jnp.concatenate([row, jnp.zeros((pad,), jnp.int32)])
        col = jnp.concatenate([col, jnp.zeros((pad,), jnp.int32)])
        ev = jnp.concatenate([ev, jnp.zeros((pad,), jnp.float32)])

    n_acc = -(-n // 2048) * 2048
    sc = _sc_edge_kernel(n, n_acc, e_pad, e_pad // (NW * CHUNK))
    partial = sc(nodes, row, col, ev, jnp.zeros((n_acc,), jnp.float32))

    summed = _tc_sum_kernel(n_acc)(partial.reshape(NC, n_acc // 128, 128))
    return summed.reshape(-1)[:n]
